# baseline (speedup 1.0000x reference)
import functools

import jax
import jax.numpy as jnp
from jax import lax
from jax.experimental import pallas as pl
from jax.experimental.pallas import tpu as pltpu
from jax.experimental.pallas import tpu_sc as plsc

_MAX_LEN = 10000
_D = 512
_B = 16384

_info = plsc.get_sparse_core_info()
_NC = _info.num_cores
_NS = _info.num_subcores
_NW = _NC * _NS
_B_PER_W = _B // _NW
_CHUNK = 16
_NCHUNK = _B_PER_W // _CHUNK
_NBUF = 12


def _gather_body(table_hbm, idx_hbm, out_hbm, idx_v, rows_v, gsem, ssem):
    wid = lax.axis_index("s") * _NC + lax.axis_index("c")
    base = wid * _B_PER_W
    pltpu.sync_copy(idx_hbm.at[pl.ds(base, _B_PER_W)], idx_v)

    def start_gather(c):
        return pltpu.async_copy(
            table_hbm.at[idx_v.at[pl.ds(c * _CHUNK, _CHUNK)]],
            rows_v.at[c % _NBUF],
            gsem.at[c % _NBUF],
        )

    def start_store(c):
        return pltpu.async_copy(
            rows_v.at[c % _NBUF],
            out_hbm.at[pl.ds(base + c * _CHUNK, _CHUNK)],
            ssem.at[c % _NBUF],
        )

    gathers = [None] * _NCHUNK
    stores = [None] * _NCHUNK
    for c in range(_NBUF):
        gathers[c] = start_gather(c)
    for c in range(_NCHUNK):
        gathers[c].wait()
        stores[c] = start_store(c)
        if c + _NBUF < _NCHUNK:
            stores[c].wait()
            gathers[c + _NBUF] = start_gather(c + _NBUF)
    for c in range(_NCHUNK - _NBUF, _NCHUNK):
        stores[c].wait()


@jax.jit
def kernel(timesteps, pos_encoding):
    mesh = plsc.VectorSubcoreMesh(core_axis_name="c", subcore_axis_name="s")
    run = functools.partial(
        pl.kernel,
        mesh=mesh,
        out_type=jax.ShapeDtypeStruct((_B, _D), jnp.float32),
        scratch_types=[
            pltpu.VMEM((_B_PER_W,), jnp.int32),
            pltpu.VMEM((_NBUF, _CHUNK, _D), jnp.float32),
            pltpu.SemaphoreType.DMA((_NBUF,)),
            pltpu.SemaphoreType.DMA((_NBUF,)),
        ],
    )(_gather_body)
    return run(pos_encoding, timesteps.astype(jnp.int32))


# speedup vs baseline: 1.0314x; 1.0314x over previous
import functools

import jax
import jax.numpy as jnp
from jax import lax
from jax.experimental import pallas as pl
from jax.experimental.pallas import tpu as pltpu
from jax.experimental.pallas import tpu_sc as plsc

_MAX_LEN = 10000
_D = 512
_B = 16384

_info = plsc.get_sparse_core_info()
_NC = _info.num_cores
_NS = _info.num_subcores
_NW = _NC * _NS
_B_PER_W = _B // _NW
_CHUNK = 32
_NCHUNK = _B_PER_W // _CHUNK
_NBUF = 6


def _gather_body(table_hbm, idx_hbm, out_hbm, idx_v, rows_v, gsem, ssem):
    wid = lax.axis_index("s") * _NC + lax.axis_index("c")
    base = wid * _B_PER_W
    pltpu.sync_copy(idx_hbm.at[pl.ds(base, _B_PER_W)], idx_v)

    def start_gather(c):
        return pltpu.async_copy(
            table_hbm.at[idx_v.at[pl.ds(c * _CHUNK, _CHUNK)]],
            rows_v.at[c % _NBUF],
            gsem.at[c % _NBUF],
        )

    def start_store(c):
        return pltpu.async_copy(
            rows_v.at[c % _NBUF],
            out_hbm.at[pl.ds(base + c * _CHUNK, _CHUNK)],
            ssem.at[c % _NBUF],
        )

    gathers = [None] * _NCHUNK
    stores = [None] * _NCHUNK
    for c in range(_NBUF):
        gathers[c] = start_gather(c)
    for c in range(_NCHUNK):
        gathers[c].wait()
        stores[c] = start_store(c)
        if c + _NBUF < _NCHUNK:
            stores[c].wait()
            gathers[c + _NBUF] = start_gather(c + _NBUF)
    for c in range(_NCHUNK - _NBUF, _NCHUNK):
        stores[c].wait()


@jax.jit
def kernel(timesteps, pos_encoding):
    mesh = plsc.VectorSubcoreMesh(core_axis_name="c", subcore_axis_name="s")
    run = functools.partial(
        pl.kernel,
        mesh=mesh,
        out_type=jax.ShapeDtypeStruct((_B, _D), jnp.float32),
        scratch_types=[
            pltpu.VMEM((_B_PER_W,), jnp.int32),
            pltpu.VMEM((_NBUF, _CHUNK, _D), jnp.float32),
            pltpu.SemaphoreType.DMA((_NBUF,)),
            pltpu.SemaphoreType.DMA((_NBUF,)),
        ],
    )(_gather_body)
    return run(pos_encoding, timesteps.astype(jnp.int32))


# speedup vs baseline: 1.0315x; 1.0001x over previous
import functools

import jax
import jax.numpy as jnp
from jax import lax
from jax.experimental import pallas as pl
from jax.experimental.pallas import tpu as pltpu
from jax.experimental.pallas import tpu_sc as plsc

_MAX_LEN = 10000
_D = 512
_B = 16384

_info = plsc.get_sparse_core_info()
_NC = _info.num_cores
_NS = _info.num_subcores
_NW = _NC * _NS
_B_PER_W = _B // _NW
_CHUNK = 32
_NCHUNK = _B_PER_W // _CHUNK
_NBUF = 6


_IDX_HEAD = _NBUF * _CHUNK


def _gather_body(table_hbm, idx_hbm, out_hbm, idx_v, rows_v, gsem, ssem, isem):
    wid = lax.axis_index("s") * _NC + lax.axis_index("c")
    base = wid * _B_PER_W
    pltpu.sync_copy(
        idx_hbm.at[pl.ds(base, _IDX_HEAD)], idx_v.at[pl.ds(0, _IDX_HEAD)]
    )
    idx_tail = pltpu.async_copy(
        idx_hbm.at[pl.ds(base + _IDX_HEAD, _B_PER_W - _IDX_HEAD)],
        idx_v.at[pl.ds(_IDX_HEAD, _B_PER_W - _IDX_HEAD)],
        isem,
    )

    def start_gather(c):
        return pltpu.async_copy(
            table_hbm.at[idx_v.at[pl.ds(c * _CHUNK, _CHUNK)]],
            rows_v.at[c % _NBUF],
            gsem.at[c % _NBUF],
        )

    def start_store(c):
        return pltpu.async_copy(
            rows_v.at[c % _NBUF],
            out_hbm.at[pl.ds(base + c * _CHUNK, _CHUNK)],
            ssem.at[c % _NBUF],
        )

    gathers = [None] * _NCHUNK
    stores = [None] * _NCHUNK
    for c in range(_NBUF):
        gathers[c] = start_gather(c)
    idx_tail.wait()
    for c in range(_NCHUNK):
        gathers[c].wait()
        stores[c] = start_store(c)
        if c + _NBUF < _NCHUNK:
            stores[c].wait()
            gathers[c + _NBUF] = start_gather(c + _NBUF)
    for c in range(_NCHUNK - _NBUF, _NCHUNK):
        stores[c].wait()


@jax.jit
def kernel(timesteps, pos_encoding):
    mesh = plsc.VectorSubcoreMesh(core_axis_name="c", subcore_axis_name="s")
    run = functools.partial(
        pl.kernel,
        mesh=mesh,
        out_type=jax.ShapeDtypeStruct((_B, _D), jnp.float32),
        scratch_types=[
            pltpu.VMEM((_B_PER_W,), jnp.int32),
            pltpu.VMEM((_NBUF, _CHUNK, _D), jnp.float32),
            pltpu.SemaphoreType.DMA((_NBUF,)),
            pltpu.SemaphoreType.DMA((_NBUF,)),
            pltpu.SemaphoreType.DMA,
        ],
    )(_gather_body)
    return run(pos_encoding, timesteps.astype(jnp.int32))


# speedup vs baseline: 1.0332x; 1.0016x over previous
import functools

import jax
import jax.numpy as jnp
from jax import lax
from jax.experimental import pallas as pl
from jax.experimental.pallas import tpu as pltpu
from jax.experimental.pallas import tpu_sc as plsc

_MAX_LEN = 10000
_D = 512
_B = 16384

_info = plsc.get_sparse_core_info()
_NC = _info.num_cores
_NS = _info.num_subcores
_NW = _NC * _NS
_B_PER_W = _B // _NW
_CHUNK = 32
_NCHUNK = _B_PER_W // _CHUNK
_NBUF = 6


def _gather_body(table_hbm, idx_hbm, out_hbm, idx_v, rows_v, gsem, ssem):
    wid = lax.axis_index("s") * _NC + lax.axis_index("c")
    base = wid * _B_PER_W
    pltpu.sync_copy(idx_hbm.at[pl.ds(base, _B_PER_W)], idx_v)

    def start_gather(c):
        return pltpu.async_copy(
            table_hbm.at[idx_v.at[pl.ds(c * _CHUNK, _CHUNK)]],
            rows_v.at[c % _NBUF],
            gsem.at[c % _NBUF],
        )

    def start_store(c):
        return pltpu.async_copy(
            rows_v.at[c % _NBUF],
            out_hbm.at[pl.ds(base + c * _CHUNK, _CHUNK)],
            ssem.at[c % _NBUF],
        )

    gathers = [None] * _NCHUNK
    stores = [None] * _NCHUNK
    for c in range(_NBUF):
        gathers[c] = start_gather(c)
    for c in range(_NCHUNK):
        gathers[c].wait()
        stores[c] = start_store(c)
        if c + _NBUF < _NCHUNK:
            stores[c].wait()
            gathers[c + _NBUF] = start_gather(c + _NBUF)
    for c in range(_NCHUNK - _NBUF, _NCHUNK):
        stores[c].wait()


@jax.jit
def kernel(timesteps, pos_encoding):
    mesh = plsc.VectorSubcoreMesh(core_axis_name="c", subcore_axis_name="s")
    run = functools.partial(
        pl.kernel,
        mesh=mesh,
        out_type=jax.ShapeDtypeStruct((_B, _D), jnp.float32),
        scratch_types=[
            pltpu.VMEM((_B_PER_W,), jnp.int32),
            pltpu.VMEM((_NBUF, _CHUNK, _D), jnp.float32),
            pltpu.SemaphoreType.DMA((_NBUF,)),
            pltpu.SemaphoreType.DMA((_NBUF,)),
        ],
    )(_gather_body)
    return run(pos_encoding, timesteps.astype(jnp.int32))
